# baseline (speedup 1.0000x reference)
import functools

import jax
import jax.numpy as jnp
from jax import lax
from jax.experimental import pallas as pl
from jax.experimental.pallas import tpu as pltpu
from jax.experimental.pallas import tpu_sc as plsc

_NW = 32
_SLAB = 3328
_NPAD = _NW * _SLAB
_GROUPS = _SLAB // 16
_HALF = _SLAB // 2
_BLK = 2048


def _route_body(ft_hbm, bt_hbm, out_hbm, f0_v, f1_v, bt_v, buf_v,
                sem1, sem2, sem3):
    wid = lax.axis_index("s") * 2 + lax.axis_index("c")
    base = wid * _SLAB
    in0 = pltpu.async_copy(ft_hbm.at[0, pl.ds(base, _SLAB)], f0_v, sem1)
    in1 = pltpu.async_copy(ft_hbm.at[1, pl.ds(base, _SLAB)], f1_v, sem2)
    in2 = pltpu.async_copy(bt_hbm.at[pl.ds(base, _SLAB)], bt_v, sem3)
    in0.wait()
    in1.wait()
    in2.wait()

    zero_f = jnp.zeros((16,), jnp.float32)
    one_f = jnp.ones((16,), jnp.float32)

    def group(k, carry):
        r = k * 16
        tv = bt_v[pl.ds(r, 16)]
        f0 = f0_v[pl.ds(r, 16)]
        f1 = f1_v[pl.ds(r, 16)]
        m1 = tv == 1
        m2 = tv == 2
        m3 = tv == 3
        buf_v[0, pl.ds(r, 16)] = jnp.where(m1, f0, zero_f)
        buf_v[1, pl.ds(r, 16)] = jnp.where(m1, f1, zero_f)
        buf_v[2, pl.ds(r, 16)] = jnp.where(m2, f0, zero_f)
        buf_v[3, pl.ds(r, 16)] = jnp.where(m2, f1, zero_f)
        buf_v[4, pl.ds(r, 16)] = jnp.where(m3, f0, zero_f)
        buf_v[5, pl.ds(r, 16)] = jnp.where(m3, f1, zero_f)
        buf_v[6, pl.ds(r, 16)] = jnp.where(m1, one_f, zero_f)
        buf_v[7, pl.ds(r, 16)] = jnp.where(m2, one_f, zero_f)
        buf_v[8, pl.ds(r, 16)] = jnp.where(m3, one_f, zero_f)
        return carry

    lax.fori_loop(0, _GROUPS // 2, group, 0)
    cp1 = pltpu.async_copy(
        buf_v.at[:, pl.ds(0, _HALF)],
        out_hbm.at[:, pl.ds(base, _HALF)], sem1)
    lax.fori_loop(_GROUPS // 2, _GROUPS, group, 0)
    cp2 = pltpu.async_copy(
        buf_v.at[:, pl.ds(_HALF, _HALF)],
        out_hbm.at[:, pl.ds(base + _HALF, _HALF)], sem2)
    cp1.wait()
    cp2.wait()


def _route(ft_pad, bt_pad):
    mesh = plsc.VectorSubcoreMesh(core_axis_name="c", subcore_axis_name="s")
    fn = functools.partial(
        pl.kernel,
        mesh=mesh,
        out_type=jax.ShapeDtypeStruct((9, _NPAD), jnp.float32),
        scratch_types=[
            pltpu.VMEM((_SLAB,), jnp.float32),
            pltpu.VMEM((_SLAB,), jnp.float32),
            pltpu.VMEM((_SLAB,), jnp.int32),
            pltpu.VMEM((9, _SLAB), jnp.float32),
            pltpu.SemaphoreType.DMA,
            pltpu.SemaphoreType.DMA,
            pltpu.SemaphoreType.DMA,
        ],
    )(_route_body)
    return fn(ft_pad, bt_pad)


def _dense_body(fc_ref, w_ref, out_ref):
    pre = lax.dot_general(
        fc_ref[...], w_ref[...],
        (((0,), (0,)), ((), ())),
        preferred_element_type=jnp.float32,
    )
    out_ref[...] = jnp.tanh(pre)


@jax.jit
def kernel(feat, btype, Ws, bs, Wg, bg, Wl, bl):
    n, _ = feat.shape
    d = Ws.shape[1]
    wcat = jnp.zeros((9, d), jnp.float32)
    wcat = wcat.at[0:2].set(Ws).at[2:4].set(Wg).at[4:6].set(Wl)
    wcat = wcat.at[6].set(bs).at[7].set(bg).at[8].set(bl)

    ft_pad = jnp.pad(feat.T, ((0, 0), (0, _NPAD - n)))
    bt_pad = jnp.pad(btype, (0, _NPAD - n))

    fct = _route(ft_pad, bt_pad)

    return pl.pallas_call(
        _dense_body,
        grid=((n + _BLK - 1) // _BLK,),
        in_specs=[
            pl.BlockSpec((9, _BLK), lambda i: (0, i)),
            pl.BlockSpec((9, d), lambda i: (0, 0)),
        ],
        out_specs=pl.BlockSpec((_BLK, d), lambda i: (i, 0)),
        out_shape=jax.ShapeDtypeStruct((n, d), jnp.float32),
    )(fct, wcat)


# speedup vs baseline: 1.1152x; 1.1152x over previous
import functools

import jax
import jax.numpy as jnp
from jax import lax
from jax.experimental import pallas as pl
from jax.experimental.pallas import tpu as pltpu
from jax.experimental.pallas import tpu_sc as plsc

_NW = 32
_SLAB = 3328
_NPAD = _NW * _SLAB
_GROUPS = _SLAB // 16
_HALF = _SLAB // 2
_BLK = 4096


def _route_body(ft_hbm, bt_hbm, out_hbm, f0_v, f1_v, bt_v, buf_v,
                sem1, sem2, sem3):
    wid = lax.axis_index("s") * 2 + lax.axis_index("c")
    base = wid * _SLAB
    in0 = pltpu.async_copy(ft_hbm.at[0, pl.ds(base, _SLAB)], f0_v, sem1)
    in1 = pltpu.async_copy(ft_hbm.at[1, pl.ds(base, _SLAB)], f1_v, sem2)
    in2 = pltpu.async_copy(bt_hbm.at[pl.ds(base, _SLAB)], bt_v, sem3)
    in0.wait()
    in1.wait()
    in2.wait()

    zero_f = jnp.zeros((16,), jnp.float32)
    one_f = jnp.ones((16,), jnp.float32)

    def group(k, carry):
        r = k * 16
        tv = bt_v[pl.ds(r, 16)]
        f0 = f0_v[pl.ds(r, 16)]
        f1 = f1_v[pl.ds(r, 16)]
        m1 = tv == 1
        m2 = tv == 2
        m3 = tv == 3
        buf_v[0, pl.ds(r, 16)] = jnp.where(m1, f0, zero_f)
        buf_v[1, pl.ds(r, 16)] = jnp.where(m1, f1, zero_f)
        buf_v[2, pl.ds(r, 16)] = jnp.where(m2, f0, zero_f)
        buf_v[3, pl.ds(r, 16)] = jnp.where(m2, f1, zero_f)
        buf_v[4, pl.ds(r, 16)] = jnp.where(m3, f0, zero_f)
        buf_v[5, pl.ds(r, 16)] = jnp.where(m3, f1, zero_f)
        buf_v[6, pl.ds(r, 16)] = jnp.where(m1, one_f, zero_f)
        buf_v[7, pl.ds(r, 16)] = jnp.where(m2, one_f, zero_f)
        buf_v[8, pl.ds(r, 16)] = jnp.where(m3, one_f, zero_f)
        return carry

    lax.fori_loop(0, _GROUPS // 2, group, 0)
    cp1 = pltpu.async_copy(
        buf_v.at[:, pl.ds(0, _HALF)],
        out_hbm.at[:, pl.ds(base, _HALF)], sem1)
    lax.fori_loop(_GROUPS // 2, _GROUPS, group, 0)
    cp2 = pltpu.async_copy(
        buf_v.at[:, pl.ds(_HALF, _HALF)],
        out_hbm.at[:, pl.ds(base + _HALF, _HALF)], sem2)
    cp1.wait()
    cp2.wait()


def _route(ft_pad, bt_pad):
    mesh = plsc.VectorSubcoreMesh(core_axis_name="c", subcore_axis_name="s")
    fn = functools.partial(
        pl.kernel,
        mesh=mesh,
        out_type=jax.ShapeDtypeStruct((9, _NPAD), jnp.float32),
        scratch_types=[
            pltpu.VMEM((_SLAB,), jnp.float32),
            pltpu.VMEM((_SLAB,), jnp.float32),
            pltpu.VMEM((_SLAB,), jnp.int32),
            pltpu.VMEM((9, _SLAB), jnp.float32),
            pltpu.SemaphoreType.DMA,
            pltpu.SemaphoreType.DMA,
            pltpu.SemaphoreType.DMA,
        ],
    )(_route_body)
    return fn(ft_pad, bt_pad)


def _dense_body(fc_ref, w_ref, out_ref):
    pre = lax.dot_general(
        fc_ref[...], w_ref[...],
        (((0,), (0,)), ((), ())),
        preferred_element_type=jnp.float32,
    )
    out_ref[...] = jnp.tanh(pre)


@jax.jit
def kernel(feat, btype, Ws, bs, Wg, bg, Wl, bl):
    n, _ = feat.shape
    d = Ws.shape[1]
    wcat = jnp.zeros((9, d), jnp.float32)
    wcat = wcat.at[0:2].set(Ws).at[2:4].set(Wg).at[4:6].set(Wl)
    wcat = wcat.at[6].set(bs).at[7].set(bg).at[8].set(bl)

    ft_pad = jnp.pad(feat.T, ((0, 0), (0, _NPAD - n)))
    bt_pad = jnp.pad(btype, (0, _NPAD - n))

    fct = _route(ft_pad, bt_pad)

    return pl.pallas_call(
        _dense_body,
        grid=((n + _BLK - 1) // _BLK,),
        in_specs=[
            pl.BlockSpec((9, _BLK), lambda i: (0, i)),
            pl.BlockSpec((9, d), lambda i: (0, 0)),
        ],
        out_specs=pl.BlockSpec((_BLK, d), lambda i: (i, 0)),
        out_shape=jax.ShapeDtypeStruct((n, d), jnp.float32),
    )(fct, wcat)
